# speedup vs baseline: 15.4144x; 15.4144x over previous
import functools

import jax
import jax.numpy as jnp
from jax import lax
from jax.experimental import pallas as pl
from jax.experimental.pallas import tpu as pltpu

N = 100000
D = 128
B = 256
R = 2000
NB = N // R
NEG = -1e30


def _body(seg_ref, feat_ref, wg_ref, wf_ref, bf_ref, out_ref, m_ref, d_ref, s_ref):
    i = pl.program_id(0)

    @pl.when(i == 0)
    def _init():
        m_ref[...] = jnp.full((1, B), NEG, jnp.float32)
        d_ref[...] = jnp.zeros((1, B), jnp.float32)
        s_ref[...] = jnp.zeros((B, D), jnp.float32)

    ids = seg_ref[0, 0, :]
    feat = feat_ref[...]
    g = jnp.sum(feat * wg_ref[0, :][None, :], axis=1)

    bidx = lax.broadcasted_iota(jnp.int32, (B, R), 0)
    onehot = ids[None, :] == bidx

    bm = jnp.max(jnp.where(onehot, g[None, :], NEG), axis=1)
    m_old = m_ref[0, :]
    m_new = jnp.maximum(m_old, bm)
    scale = jnp.exp(m_old - m_new)

    m_row = jnp.max(jnp.where(onehot, m_new[:, None], NEG), axis=0)
    e = jnp.exp(g - m_row)

    pf = onehot.astype(jnp.float32)
    d_new = scale * d_ref[0, :] + jnp.sum(pf * e[None, :], axis=1)
    s_new = scale[:, None] * s_ref[...] + lax.dot_general(
        pf, e[:, None] * feat, (((1,), (0,)), ((), ())),
        preferred_element_type=jnp.float32)

    m_ref[0, :] = m_new
    d_ref[0, :] = d_new
    s_ref[...] = s_new

    @pl.when(i == NB - 1)
    def _finish():
        d = d_new
        nonempty = d > 0.0
        a = s_new / jnp.where(nonempty, d, 1.0)[:, None]
        out = lax.dot_general(a, wf_ref[...], (((1,), (0,)), ((), ())),
                              preferred_element_type=jnp.float32)
        out_ref[...] = out + jnp.where(nonempty, 1.0, 0.0)[:, None] * bf_ref[0, :][None, :]


@jax.jit
def _run(feat, seg_r, wg_row, W_feat, bf_row):
    grid = (NB,)
    return pl.pallas_call(
        _body,
        grid=grid,
        in_specs=[
            pl.BlockSpec((1, 1, R), lambda i: (i, 0, 0)),
            pl.BlockSpec((R, D), lambda i: (i, 0)),
            pl.BlockSpec((1, D), lambda i: (0, 0)),
            pl.BlockSpec((D, D), lambda i: (0, 0)),
            pl.BlockSpec((1, D), lambda i: (0, 0)),
        ],
        out_specs=pl.BlockSpec((B, D), lambda i: (0, 0)),
        out_shape=jax.ShapeDtypeStruct((B, D), jnp.float32),
        scratch_shapes=[
            pltpu.VMEM((1, B), jnp.float32),
            pltpu.VMEM((1, B), jnp.float32),
            pltpu.VMEM((B, D), jnp.float32),
        ],
    )(seg_r, feat, wg_row, W_feat, bf_row)


def kernel(feat, segment_ids, W_gate, b_gate, W_feat, b_feat):
    seg = segment_ids.astype(jnp.int32).reshape(NB, 1, R)
    wg_row = W_gate.reshape(1, D)
    bf_row = b_feat.reshape(1, D)
    return _run(feat, seg, wg_row, W_feat, bf_row)
